# baseline (speedup 1.0000x reference)
import functools
import math

import jax
import jax.numpy as jnp
from jax import lax
from jax.experimental import pallas as pl
from jax.experimental.pallas import tpu as pltpu, tpu_sc as plsc

N_PTS = 524288
N_LEVELS = 16
F = 2
T = 2 ** 19
MASK = T - 1
BASE_RES = 16
MAX_RES = 2048
_SCALE = math.exp(math.log(MAX_RES / BASE_RES) / (N_LEVELS - 1))
RES = [int(math.floor(BASE_RES * _SCALE ** l)) for l in range(N_LEVELS)]
DENSE = [(r + 1) ** 3 <= T for r in RES]
P1 = -1640531535
P2 = 805459861

NC, NS = 2, 16
NW = NC * NS
C = 512
G = C // 16
NPW = N_PTS // NW
NCHUNK = NPW // C


def _body(x_hbm, tab_hbm, out_hbm, xinbuf, fracbuf, idxbuf, exbuf, gath,
          outc, sem):
    wid = lax.axis_index("s") * NC + lax.axis_index("c")
    iota = lax.iota(jnp.int32, 16)
    dupa = lax.shift_right_logical(iota, 1)
    dupb = dupa + 8
    parity = lax.bitwise_and(iota, 1)
    pbase0 = wid * NPW

    def chunk_body(ci, carry):
        pbase = pbase0 + ci * C
        for d in range(3):
            pltpu.sync_copy(x_hbm.at[pl.ds(d * N_PTS + pbase, C)],
                            xinbuf.at[pl.ds(d * C, C)])

        def p0(g, carry0):
            off = g * 16
            for d in range(3):
                xd = xinbuf[pl.ds(d * C + off, 16)]
                xinbuf[pl.ds(d * C + off, 16)] = (xd + 1.0) * 0.5
            return carry0

        lax.fori_loop(0, G, p0, 0)

        for l in range(N_LEVELS):
            resf = float(RES[l])
            lT = l * T

            def pA(g, carryA, l=l, resf=resf, lT=lT):
                off = g * 16
                pis = []
                for d in range(3):
                    pos = xinbuf[pl.ds(d * C + off, 16)] * resf
                    pi = pos.astype(jnp.int32)
                    fracbuf[pl.ds(d * C + off, 16)] = pos - pi.astype(
                        jnp.float32)
                    pis.append(pi)
                px = (pis[0], pis[0] + 1)
                py = (pis[1], pis[1] + 1)
                pz = (pis[2], pis[2] + 1)
                if DENSE[l]:
                    s1 = RES[l] + 1
                    s2 = s1 * s1
                    cy = (py[0] * s1, py[1] * s1)
                    cz = (pz[0] * s2 + lT, pz[1] * s2 + lT)
                    gidxs = [
                        px[c & 1] + cy[(c >> 1) & 1] + cz[(c >> 2) & 1]
                        for c in range(8)
                    ]
                else:
                    hy = (py[0] * P1, py[1] * P1)
                    hz = (pz[0] * P2, pz[1] * P2)
                    gidxs = [
                        ((px[c & 1] ^ hy[(c >> 1) & 1] ^ hz[(c >> 2) & 1])
                         & MASK) + lT
                        for c in range(8)
                    ]
                for c, gidx in enumerate(gidxs):
                    idxbuf[pl.ds(c * C + off, 16)] = lax.shift_right_logical(
                        gidx, 2)
                    exbuf[pl.ds(c * C + off, 16)] = lax.shift_left(
                        lax.bitwise_and(gidx, 3), 1)
                return carryA

            lax.fori_loop(0, G, pA, 0)

            pltpu.async_copy(tab_hbm.at[idxbuf], gath, sem).wait()

            def pB(g, carryB, l=l):
                off = g * 16
                rowa = off + dupa
                rowb = off + dupb
                fr = []
                for half_rows in (rowa, rowb):
                    fr.append([
                        plsc.load_gather(fracbuf, [d * C + half_rows])
                        for d in range(3)
                    ])
                vals = []
                for c in range(8):
                    ra = c * C + rowa
                    rb = c * C + rowb
                    ea = plsc.load_gather(exbuf, [ra]) + parity
                    eb = plsc.load_gather(exbuf, [rb]) + parity
                    va = plsc.load_gather(gath, [ra, ea])
                    vb = plsc.load_gather(gath, [rb, eb])
                    vals.append((va, vb))

                def lerp(a, b, t):
                    return a + (b - a) * t

                cols = parity + 2 * l
                for h, rows in ((0, rowa), (1, rowb)):
                    fx, fy, fz = fr[h]
                    v01 = lerp(vals[0][h], vals[1][h], fx)
                    v23 = lerp(vals[2][h], vals[3][h], fx)
                    v45 = lerp(vals[4][h], vals[5][h], fx)
                    v67 = lerp(vals[6][h], vals[7][h], fx)
                    v0 = lerp(v01, v23, fy)
                    v1 = lerp(v45, v67, fy)
                    plsc.store_scatter(
                        outc, [lax.shift_left(rows, 5) + cols],
                        lerp(v0, v1, fz))
                return carryB

            lax.fori_loop(0, G, pB, 0)

        pltpu.sync_copy(outc, out_hbm.at[pl.ds(pbase * 32, C * 32)])
        return carry

    lax.fori_loop(0, NCHUNK, chunk_body, 0)


_hashgrid_sc = pl.kernel(
    _body,
    out_type=jax.ShapeDtypeStruct((N_PTS * N_LEVELS * F,), jnp.float32),
    mesh=plsc.VectorSubcoreMesh(core_axis_name="c", subcore_axis_name="s"),
    scratch_types=[
        pltpu.VMEM((3 * C,), jnp.float32),
        pltpu.VMEM((3 * C,), jnp.float32),
        pltpu.VMEM((8 * C,), jnp.int32),
        pltpu.VMEM((8 * C,), jnp.int32),
        pltpu.VMEM((8 * C, 8), jnp.float32),
        pltpu.VMEM((C * N_LEVELS * F,), jnp.float32),
        pltpu.SemaphoreType.DMA,
    ],
    compiler_params=pltpu.CompilerParams(
        needs_layout_passes=False, use_tc_tiling_on_sc=False),
)


def kernel(x, table):
    xt = x.T.reshape(3 * N_PTS)
    tab8 = table.reshape(N_LEVELS * T * F // 8, 8)
    return _hashgrid_sc(xt, tab8).reshape(N_PTS, N_LEVELS * F)


# speedup vs baseline: 2.7497x; 2.7497x over previous
import functools
import math

import jax
import jax.numpy as jnp
from jax import lax
from jax.experimental import pallas as pl
from jax.experimental.pallas import tpu as pltpu, tpu_sc as plsc

N_PTS = 524288
N_LEVELS = 16
F = 2
T = 2 ** 19
MASK = T - 1
BASE_RES = 16
MAX_RES = 2048
_SCALE = math.exp(math.log(MAX_RES / BASE_RES) / (N_LEVELS - 1))
RES = [int(math.floor(BASE_RES * _SCALE ** l)) for l in range(N_LEVELS)]
DENSE = [(r + 1) ** 3 <= T for r in RES]
P1 = -1640531535
P2 = 805459861

NC, NS = 2, 16
NW = NC * NS
C = 256
G = C // 16
NPW = N_PTS // NW
NCHUNK = NPW // C


def _body(x_hbm, tab_hbm, out_hbm, xinbuf, fracbuf0, fracbuf1, idxbuf0,
          idxbuf1, exbuf0, exbuf1, gath0, gath1, outc, sem0, sem1):
    wid = lax.axis_index("s") * NC + lax.axis_index("c")
    iota = lax.iota(jnp.int32, 16)
    pbase0 = wid * NPW
    fracbufs = (fracbuf0, fracbuf1)
    idxbufs = (idxbuf0, idxbuf1)
    exbufs = (exbuf0, exbuf1)
    gaths = (gath0, gath1)
    sems = (sem0, sem1)

    def chunk_body(ci, carry):
        pbase = pbase0 + ci * C
        for d in range(3):
            pltpu.sync_copy(x_hbm.at[pl.ds(d * N_PTS + pbase, C)],
                            xinbuf.at[pl.ds(d * C, C)])

        def p0(g, carry0):
            off = g * 16
            for d in range(3):
                xd = xinbuf[pl.ds(d * C + off, 16)]
                xinbuf[pl.ds(d * C + off, 16)] = (xd + 1.0) * 0.5
            return carry0

        lax.fori_loop(0, G, p0, 0)

        def pA(l, b):
            resf = float(RES[l])
            lT17 = l * (T // 4)
            fracbuf, idxbuf, exbuf = fracbufs[b], idxbufs[b], exbufs[b]

            def body_g(g, carryA):
                off = g * 16
                pis = []
                for d in range(3):
                    pos = xinbuf[pl.ds(d * C + off, 16)] * resf
                    pi = pos.astype(jnp.int32)
                    fracbuf[pl.ds(d * C + off, 16)] = pos - pi.astype(
                        jnp.float32)
                    pis.append(pi)
                px = (pis[0], pis[0] + 1)
                py = (pis[1], pis[1] + 1)
                pz = (pis[2], pis[2] + 1)
                if DENSE[l]:
                    s1 = RES[l] + 1
                    s2 = s1 * s1
                    cy = (py[0] * s1, py[1] * s1)
                    cz = (pz[0] * s2, pz[1] * s2)
                    es = [
                        px[c & 1] + cy[(c >> 1) & 1] + cz[(c >> 2) & 1]
                        for c in range(8)
                    ]
                else:
                    hy = (py[0] * P1, py[1] * P1)
                    hz = (pz[0] * P2, pz[1] * P2)
                    es = [
                        (px[c & 1] ^ hy[(c >> 1) & 1] ^ hz[(c >> 2) & 1])
                        & MASK
                        for c in range(8)
                    ]
                for c, e in enumerate(es):
                    q0 = (lT17 + lax.shift_left(
                        lax.shift_right_logical(e, 7), 5)
                        + lax.bitwise_and(lax.shift_right_logical(e, 3), 15))
                    idxbuf[pl.ds((2 * c) * C + off, 16)] = q0
                    idxbuf[pl.ds((2 * c + 1) * C + off, 16)] = q0 + 16
                    exbuf[pl.ds(c * C + off, 16)] = lax.bitwise_and(e, 7)
                return carryA

            lax.fori_loop(0, G, body_g, 0)
            return pltpu.async_copy(tab_hbm.at[idxbufs[b]], gaths[b],
                                    sems[b])

        def pB(l, b):
            fracbuf, exbuf, gath = fracbufs[b], exbufs[b], gaths[b]

            def body_g(g, carryB):
                off = g * 16
                rows = off + iota
                fx = fracbuf[pl.ds(0 * C + off, 16)]
                fy = fracbuf[pl.ds(1 * C + off, 16)]
                fz = fracbuf[pl.ds(2 * C + off, 16)]
                vals = []
                for c in range(8):
                    exv = exbuf[pl.ds(c * C + off, 16)]
                    v0 = plsc.load_gather(gath, [(2 * c) * C + rows, exv])
                    v1 = plsc.load_gather(gath, [(2 * c + 1) * C + rows, exv])
                    vals.append((v0, v1))

                def lerp(a, b_, t):
                    return a + (b_ - a) * t

                obase = lax.shift_left(rows, 5) + 2 * l
                for f in range(2):
                    v01 = lerp(vals[0][f], vals[1][f], fx)
                    v23 = lerp(vals[2][f], vals[3][f], fx)
                    v45 = lerp(vals[4][f], vals[5][f], fx)
                    v67 = lerp(vals[6][f], vals[7][f], fx)
                    v0 = lerp(v01, v23, fy)
                    v1 = lerp(v45, v67, fy)
                    plsc.store_scatter(outc, [obase + f], lerp(v0, v1, fz))
                return carryB

            lax.fori_loop(0, G, body_g, 0)

        handle = pA(0, 0)
        for l in range(N_LEVELS):
            b = l % 2
            nxt = pA(l + 1, 1 - b) if l + 1 < N_LEVELS else None
            handle.wait()
            pB(l, b)
            handle = nxt

        pltpu.sync_copy(outc, out_hbm.at[pl.ds(pbase * 32, C * 32)])
        return carry

    lax.fori_loop(0, NCHUNK, chunk_body, 0)


_hashgrid_sc = pl.kernel(
    _body,
    out_type=jax.ShapeDtypeStruct((N_PTS * N_LEVELS * F,), jnp.float32),
    mesh=plsc.VectorSubcoreMesh(core_axis_name="c", subcore_axis_name="s"),
    scratch_types=[
        pltpu.VMEM((3 * C,), jnp.float32),
        pltpu.VMEM((3 * C,), jnp.float32),
        pltpu.VMEM((3 * C,), jnp.float32),
        pltpu.VMEM((16 * C,), jnp.int32),
        pltpu.VMEM((16 * C,), jnp.int32),
        pltpu.VMEM((8 * C,), jnp.int32),
        pltpu.VMEM((8 * C,), jnp.int32),
        pltpu.VMEM((16 * C, 8), jnp.float32),
        pltpu.VMEM((16 * C, 8), jnp.float32),
        pltpu.VMEM((C * N_LEVELS * F,), jnp.float32),
        pltpu.SemaphoreType.DMA,
        pltpu.SemaphoreType.DMA,
    ],
    compiler_params=pltpu.CompilerParams(
        needs_layout_passes=False, use_tc_tiling_on_sc=False),
)


def kernel(x, table):
    xt = x.T.reshape(3 * N_PTS)
    tab8 = (table.reshape(N_LEVELS, T // 128, 128, F)
            .transpose(0, 1, 3, 2)
            .reshape(N_LEVELS * T * F // 8, 8))
    return _hashgrid_sc(xt, tab8).reshape(N_PTS, N_LEVELS * F)


# speedup vs baseline: 5.0053x; 1.8203x over previous
import functools
import math

import jax
import jax.numpy as jnp
from jax import lax
from jax.experimental import pallas as pl
from jax.experimental.pallas import tpu as pltpu, tpu_sc as plsc

N_PTS = 524288
N_LEVELS = 16
F = 2
T = 2 ** 19
MASK = T - 1
BASE_RES = 16
MAX_RES = 2048
_SCALE = math.exp(math.log(MAX_RES / BASE_RES) / (N_LEVELS - 1))
RES = [int(math.floor(BASE_RES * _SCALE ** l)) for l in range(N_LEVELS)]
DENSE = [(r + 1) ** 3 <= T for r in RES]
P1 = -1640531535
P2 = 805459861

NC, NS = 2, 16
NW = NC * NS
C = 512
G = C // 16
NPW = N_PTS // NW
NCHUNK = NPW // C

NWORDS = N_LEVELS * T * F
BLK = 256
NBLK = NWORDS // BLK
BLK_PER_W = NBLK // NW
SLAB = 64
NSLAB = BLK_PER_W // SLAB


def _relayout_body(tabn_hbm, out_hbm, inslab, outslab, sem):
    wid = lax.axis_index("s") * NC + lax.axis_index("c")
    iota = lax.iota(jnp.int32, 16)
    iota2 = lax.shift_left(iota, 1)
    wbase = wid * BLK_PER_W * BLK

    def slab_body(s, carry):
        base = wbase + s * (SLAB * BLK)
        pltpu.sync_copy(tabn_hbm.at[pl.ds(base, SLAB * BLK)], inslab)

        def grp(k, c2):
            boff = lax.shift_left(lax.shift_right_logical(k, 3), 8)
            goff = lax.shift_left(lax.bitwise_and(k, 7), 4)
            v0 = inslab[pl.ds(boff + goff, 16)]
            v1 = inslab[pl.ds(boff + 128 + goff, 16)]
            obase = boff + lax.shift_left(goff, 1) + iota2
            plsc.store_scatter(outslab, [obase], v0)
            plsc.store_scatter(outslab, [obase + 1], v1)
            return c2

        lax.fori_loop(0, SLAB * 8, grp, 0)
        pltpu.sync_copy(outslab, out_hbm.at[pl.ds(base, SLAB * BLK)])
        return carry

    lax.fori_loop(0, NSLAB, slab_body, 0)


_relayout_sc = pl.kernel(
    _relayout_body,
    out_type=jax.ShapeDtypeStruct((NWORDS,), jnp.float32),
    mesh=plsc.VectorSubcoreMesh(core_axis_name="c", subcore_axis_name="s"),
    scratch_types=[
        pltpu.VMEM((SLAB * BLK,), jnp.float32),
        pltpu.VMEM((SLAB * BLK,), jnp.float32),
        pltpu.SemaphoreType.DMA,
    ],
    compiler_params=pltpu.CompilerParams(
        needs_layout_passes=False, use_tc_tiling_on_sc=False),
)


def _body(x_hbm, tab_hbm, out_hbm, xinbuf, fracbuf0, fracbuf1, idxbuf0,
          idxbuf1, exbuf0, exbuf1, gath0, gath1, outc, sem0, sem1):
    wid = lax.axis_index("s") * NC + lax.axis_index("c")
    iota = lax.iota(jnp.int32, 16)
    pbase0 = wid * NPW
    fracbufs = (fracbuf0, fracbuf1)
    idxbufs = (idxbuf0, idxbuf1)
    exbufs = (exbuf0, exbuf1)
    gaths = (gath0, gath1)
    sems = (sem0, sem1)

    def chunk_body(ci, carry):
        pbase = pbase0 + ci * C
        for d in range(3):
            pltpu.sync_copy(x_hbm.at[pl.ds(d * N_PTS + pbase, C)],
                            xinbuf.at[pl.ds(d * C, C)])

        def p0(g, carry0):
            off = g * 16
            for d in range(3):
                xd = xinbuf[pl.ds(d * C + off, 16)]
                xinbuf[pl.ds(d * C + off, 16)] = (xd + 1.0) * 0.5
            return carry0

        lax.fori_loop(0, G, p0, 0)

        def pA(l, b):
            resf = float(RES[l])
            lT = l * T
            fracbuf, idxbuf, exbuf = fracbufs[b], idxbufs[b], exbufs[b]

            def body_g(g, carryA):
                off = g * 16
                pis = []
                for d in range(3):
                    pos = xinbuf[pl.ds(d * C + off, 16)] * resf
                    pi = pos.astype(jnp.int32)
                    fracbuf[pl.ds(d * C + off, 16)] = pos - pi.astype(
                        jnp.float32)
                    pis.append(pi)
                px = (pis[0], pis[0] + 1)
                py = (pis[1], pis[1] + 1)
                pz = (pis[2], pis[2] + 1)
                if DENSE[l]:
                    s1 = RES[l] + 1
                    s2 = s1 * s1
                    cy = (py[0] * s1, py[1] * s1)
                    cz = (pz[0] * s2 + lT, pz[1] * s2 + lT)
                    es = [
                        px[c & 1] + cy[(c >> 1) & 1] + cz[(c >> 2) & 1]
                        for c in range(8)
                    ]
                else:
                    hy = (py[0] * P1, py[1] * P1)
                    hz = (pz[0] * P2, pz[1] * P2)
                    es = [
                        ((px[c & 1] ^ hy[(c >> 1) & 1] ^ hz[(c >> 2) & 1])
                         & MASK) + lT
                        for c in range(8)
                    ]
                for c, gidx in enumerate(es):
                    idxbuf[pl.ds(c * C + off, 16)] = lax.shift_right_logical(
                        gidx, 2)
                    exbuf[pl.ds(c * C + off, 16)] = lax.shift_left(
                        lax.bitwise_and(gidx, 3), 1)
                return carryA

            lax.fori_loop(0, G, body_g, 0)
            return pltpu.async_copy(tab_hbm.at[idxbufs[b]], gaths[b],
                                    sems[b])

        def pB(l, b):
            fracbuf, exbuf, gath = fracbufs[b], exbufs[b], gaths[b]

            def body_g(g, carryB):
                off = g * 16
                rows = off + iota
                fx = fracbuf[pl.ds(0 * C + off, 16)]
                fy = fracbuf[pl.ds(1 * C + off, 16)]
                fz = fracbuf[pl.ds(2 * C + off, 16)]
                vals = []
                for c in range(8):
                    exv = exbuf[pl.ds(c * C + off, 16)]
                    crows = c * C + rows
                    v0 = plsc.load_gather(gath, [crows, exv])
                    v1 = plsc.load_gather(gath, [crows, exv + 1])
                    vals.append((v0, v1))

                def lerp(a, b_, t):
                    return a + (b_ - a) * t

                obase = lax.shift_left(rows, 5) + 2 * l
                for f in range(2):
                    v01 = lerp(vals[0][f], vals[1][f], fx)
                    v23 = lerp(vals[2][f], vals[3][f], fx)
                    v45 = lerp(vals[4][f], vals[5][f], fx)
                    v67 = lerp(vals[6][f], vals[7][f], fx)
                    v0 = lerp(v01, v23, fy)
                    v1 = lerp(v45, v67, fy)
                    plsc.store_scatter(outc, [obase + f], lerp(v0, v1, fz))
                return carryB

            lax.fori_loop(0, G, body_g, 0)

        handle = pA(0, 0)
        for l in range(N_LEVELS):
            b = l % 2
            nxt = pA(l + 1, 1 - b) if l + 1 < N_LEVELS else None
            handle.wait()
            pB(l, b)
            handle = nxt

        pltpu.sync_copy(outc, out_hbm.at[pl.ds(pbase * 32, C * 32)])
        return carry

    lax.fori_loop(0, NCHUNK, chunk_body, 0)


_hashgrid_sc = pl.kernel(
    _body,
    out_type=jax.ShapeDtypeStruct((N_PTS * N_LEVELS * F,), jnp.float32),
    mesh=plsc.VectorSubcoreMesh(core_axis_name="c", subcore_axis_name="s"),
    scratch_types=[
        pltpu.VMEM((3 * C,), jnp.float32),
        pltpu.VMEM((3 * C,), jnp.float32),
        pltpu.VMEM((3 * C,), jnp.float32),
        pltpu.VMEM((8 * C,), jnp.int32),
        pltpu.VMEM((8 * C,), jnp.int32),
        pltpu.VMEM((8 * C,), jnp.int32),
        pltpu.VMEM((8 * C,), jnp.int32),
        pltpu.VMEM((8 * C, 8), jnp.float32),
        pltpu.VMEM((8 * C, 8), jnp.float32),
        pltpu.VMEM((C * N_LEVELS * F,), jnp.float32),
        pltpu.SemaphoreType.DMA,
        pltpu.SemaphoreType.DMA,
    ],
    compiler_params=pltpu.CompilerParams(
        needs_layout_passes=False, use_tc_tiling_on_sc=False),
)


def kernel(x, table):
    xt = x.T.reshape(3 * N_PTS)
    tabn = (table.reshape(N_LEVELS, T // 128, 128, F)
            .transpose(0, 1, 3, 2)
            .reshape(NWORDS))
    tab_em = _relayout_sc(tabn).reshape(NWORDS // 8, 8)
    return _hashgrid_sc(xt, tab_em).reshape(N_PTS, N_LEVELS * F)
